# baseline (speedup 1.0000x reference)
import functools
import jax
import jax.numpy as jnp
import numpy as np
from jax.experimental import pallas as pl
from jax.experimental.pallas import tpu as pltpu
from jax.experimental.pallas import tpu_sc as plsc

B = 8
N0 = 2048
M0 = 1024
M1 = 256
K = 64
R0 = 0.2
R1 = 0.4
BN_EPS = 1e-5
INF = jnp.inf


def _fps_body(px_ref, py_ref, pz_ref, cx_ref, cy_ref, cz_ref, *, m):
    px = px_ref[...]
    py = py_ref[...]
    pz = pz_ref[...]
    n = px.shape[1]
    lane_n = jax.lax.broadcasted_iota(jnp.int32, (B, n), 1)
    lane_m = jax.lax.broadcasted_iota(jnp.int32, (B, m), 1)

    def body(i, carry):
        dists, cur, ax, ay, az = carry
        mask = lane_n == cur
        cx = jnp.max(jnp.where(mask, px, -INF), axis=1, keepdims=True)
        cy = jnp.max(jnp.where(mask, py, -INF), axis=1, keepdims=True)
        cz = jnp.max(jnp.where(mask, pz, -INF), axis=1, keepdims=True)
        smask = lane_m == i
        ax = jnp.where(smask, cx, ax)
        ay = jnp.where(smask, cy, ay)
        az = jnp.where(smask, cz, az)
        d = (px - cx) ** 2 + (py - cy) ** 2 + (pz - cz) ** 2
        dists = jnp.minimum(dists, d)
        mx = jnp.max(dists, axis=1, keepdims=True)
        cur = jnp.min(jnp.where(dists == mx, lane_n, n), axis=1, keepdims=True)
        return dists, cur, ax, ay, az

    init = (
        jnp.full((B, n), INF, jnp.float32),
        jnp.zeros((B, 1), jnp.int32),
        jnp.zeros((B, m), jnp.float32),
        jnp.zeros((B, m), jnp.float32),
        jnp.zeros((B, m), jnp.float32),
    )
    _, _, ax, ay, az = jax.lax.fori_loop(0, m, body, init)
    cx_ref[...] = ax
    cy_ref[...] = ay
    cz_ref[...] = az


def _fps(px, py, pz, m):
    out = jax.ShapeDtypeStruct((B, m), jnp.float32)
    return pl.pallas_call(
        functools.partial(_fps_body, m=m),
        out_shape=(out, out, out),
    )(px, py, pz)


def _nbr_body(px_ref, py_ref, pz_ref, cx_ref, cy_ref, cz_ref, pack_ref,
              idx_ref, *, r2, n, m):
    b = pl.program_id(0)
    px = px_ref[pl.ds(b, 1), :]
    py = py_ref[pl.ds(b, 1), :]
    pz = pz_ref[pl.ds(b, 1), :]
    lane_b = jax.lax.broadcasted_iota(jnp.int32, (m, B), 1)
    colmask = lane_b == b
    cx = jnp.sum(jnp.where(colmask, cx_ref[...], 0.0), axis=1, keepdims=True)
    cy = jnp.sum(jnp.where(colmask, cy_ref[...], 0.0), axis=1, keepdims=True)
    cz = jnp.sum(jnp.where(colmask, cz_ref[...], 0.0), axis=1, keepdims=True)
    d2 = (cx - px) ** 2 + (cy - py) ** 2 + (cz - pz) ** 2
    d2 = jnp.where(d2 <= r2, d2, INF)

    def rbody(i, t):
        bb = 29 - 2 * i
        hi = jax.lax.shift_left(jnp.int32(1), bb)
        lo = jax.lax.shift_left(jnp.int32(1), bb - 1)
        u00 = t | (lo - 1)
        u01 = u00 | lo
        u10 = u00 | hi
        c00 = jnp.sum(jnp.where(
            d2 <= jax.lax.bitcast_convert_type(u00, jnp.float32), 1.0, 0.0),
            axis=1, keepdims=True)
        c01 = jnp.sum(jnp.where(
            d2 <= jax.lax.bitcast_convert_type(u01, jnp.float32), 1.0, 0.0),
            axis=1, keepdims=True)
        c10 = jnp.sum(jnp.where(
            d2 <= jax.lax.bitcast_convert_type(u10, jnp.float32), 1.0, 0.0),
            axis=1, keepdims=True)
        kf = float(K)
        add = jnp.where(
            c00 >= kf, 0,
            jnp.where(c01 >= kf, lo, jnp.where(c10 >= kf, hi, hi | lo)))
        return t | add

    t = jax.lax.fori_loop(0, 15, rbody, jnp.zeros((m, 1), jnp.int32))
    tf = jax.lax.bitcast_convert_type(t, jnp.float32)
    selm = jnp.where(d2 <= tf, 1.0, 0.0)
    words = jnp.dot(selm, pack_ref[...],
                    preferred_element_type=jnp.float32).astype(jnp.int32)
    nw = n // 16
    lane_w = jax.lax.broadcasted_iota(jnp.int32, (m, nw), 1)
    lane_k = jax.lax.broadcasted_iota(jnp.int32, (m, K), 1)

    def ebody(s, carry):
        words, first, acc = carry
        nz = words != 0
        fl = jnp.min(jnp.where(nz, lane_w, nw), axis=1, keepdims=True)
        has = fl < nw
        onfl = lane_w == fl
        w = jnp.max(jnp.where(onfl, words, 0), axis=1, keepdims=True)
        low = jnp.bitwise_and(w, -w)
        bidx = jax.lax.shift_right_logical(
            jax.lax.bitcast_convert_type(low.astype(jnp.float32), jnp.int32),
            23) - 127
        idxv = fl * 16 + bidx
        first = jnp.where(s == 0, idxv, first)
        selv = jnp.where(has, idxv, first)
        acc = jnp.where(lane_k == s, selv, acc)
        words = jnp.where(onfl, jnp.bitwise_and(words, words - 1), words)
        return words, first, acc

    _, _, acc = jax.lax.fori_loop(
        0, K, ebody,
        (words, jnp.zeros((m, 1), jnp.int32), jnp.zeros((m, K), jnp.int32)))
    idx_ref[0] = acc + b * n


def _nbr(px, py, pz, cxt, cyt, czt, r, n, m):
    nw = n // 16
    packw = np.zeros((n, nw), np.float32)
    packw[np.arange(n), np.arange(n) // 16] = 2.0 ** (np.arange(n) % 16)
    return pl.pallas_call(
        functools.partial(_nbr_body, r2=np.float32(r * r), n=n, m=m),
        grid=(B,),
        in_specs=[
            pl.BlockSpec((B, n), lambda b: (0, 0)),
            pl.BlockSpec((B, n), lambda b: (0, 0)),
            pl.BlockSpec((B, n), lambda b: (0, 0)),
            pl.BlockSpec((m, B), lambda b: (0, 0)),
            pl.BlockSpec((m, B), lambda b: (0, 0)),
            pl.BlockSpec((m, B), lambda b: (0, 0)),
            pl.BlockSpec((n, nw), lambda b: (0, 0)),
        ],
        out_specs=pl.BlockSpec((1, m, K), lambda b: (b, 0, 0)),
        out_shape=jax.ShapeDtypeStruct((B, m, K), jnp.int32),
    )(px, py, pz, cxt, cyt, czt, jnp.asarray(packw))


def _sc_gather(table, indices):
    num = indices.shape[0]
    cdim = table.shape[1]
    inds = indices.reshape(1, num)
    mesh = plsc.VectorSubcoreMesh(core_axis_name="c", subcore_axis_name="s")
    win = 128

    @pl.kernel(out_type=jax.ShapeDtypeStruct((num, cdim), table.dtype),
               mesh=mesh)
    def kern(x_hbm, i_hbm, o_hbm):
        def body(i_vmem, o_vmem):
            pltpu.sync_copy(x_hbm.at[i_vmem.at[0]], o_vmem)

        pltpu.emit_pipeline(
            body,
            grid=(num // win,),
            in_specs=[pl.BlockSpec((1, win), index_map=lambda i: (0, i))],
            out_specs=[pl.BlockSpec((win, cdim), index_map=lambda i: (i, 0))],
            core_axis_name="s",
            dimension_semantics=(pltpu.PARALLEL,),
        )(i_hbm, o_hbm)

    return kern(table, inds)


def _proj_body(x_ref, y_ref, z_ref, w_ref, b_ref, out_ref):
    w = w_ref[...]
    acc = (x_ref[...] * w[0:1, :] + y_ref[...] * w[1:2, :]
           + z_ref[...] * w[2:3, :] + b_ref[...])
    out_ref[...] = acc


def _proj(xc, yc, zc, w, b):
    n = xc.shape[0]
    c = w.shape[1]
    return pl.pallas_call(
        _proj_body,
        out_shape=jax.ShapeDtypeStruct((n, c), jnp.float32),
    )(xc, yc, zc, w, b.reshape(1, c))


def _feat_proj_body(x_ref, w_ref, px_ref, py_ref, pz_ref, wp_ref, b_ref,
                    out_ref):
    wp = wp_ref[...]
    acc = jnp.dot(x_ref[...], w_ref[...], preferred_element_type=jnp.float32)
    acc += (px_ref[...] * wp[0:1, :] + py_ref[...] * wp[1:2, :]
            + pz_ref[...] * wp[2:3, :] + b_ref[...])
    out_ref[...] = acc


def _feat_proj(x, w, px, py, pz, wp, b):
    n, c = x.shape[0], w.shape[1]
    return pl.pallas_call(
        _feat_proj_body,
        out_shape=jax.ShapeDtypeStruct((n, c), jnp.float32),
    )(x, w, px, py, pz, wp, b.reshape(1, c))


def _pairmlp(g3, c3, w2, b2, s1, t1, s2, t2, tm):
    m, k, ch = g3.shape
    c2 = w2.shape[1]

    def body(g_ref, c_ref, w2_ref, b2_ref, s1_ref, t1_ref, s2_ref, t2_ref,
             out_ref):
        g = g_ref[...]
        c = c_ref[...]
        h = jnp.maximum(g - c, 0.0) * s1_ref[...] + t1_ref[...]
        h2 = jnp.dot(h.reshape(tm * k, ch), w2_ref[...],
                     preferred_element_type=jnp.float32) + b2_ref[...]
        h2 = jnp.maximum(h2, 0.0) * s2_ref[...] + t2_ref[...]
        out_ref[...] = jnp.max(h2.reshape(tm, k, c2), axis=1)

    return pl.pallas_call(
        body,
        grid=(m // tm,),
        in_specs=[
            pl.BlockSpec((tm, k, ch), lambda i: (i, 0, 0)),
            pl.BlockSpec((tm, 1, ch), lambda i: (i, 0, 0)),
            pl.BlockSpec((ch, c2), lambda i: (0, 0)),
            pl.BlockSpec((1, c2), lambda i: (0, 0)),
            pl.BlockSpec((1, 1, ch), lambda i: (0, 0, 0)),
            pl.BlockSpec((1, 1, ch), lambda i: (0, 0, 0)),
            pl.BlockSpec((1, c2), lambda i: (0, 0)),
            pl.BlockSpec((1, c2), lambda i: (0, 0)),
        ],
        out_specs=pl.BlockSpec((tm, c2), lambda i: (i, 0)),
        out_shape=jax.ShapeDtypeStruct((m, c2), jnp.float32),
    )(g3, c3, w2, b2.reshape(1, c2), s1.reshape(1, 1, ch),
      t1.reshape(1, 1, ch), s2.reshape(1, c2), t2.reshape(1, c2))


def _head_kernel(feat_ref, wg_ref, bg_ref, gg_ref, betag_ref,
                 w0_ref, b0_ref, w1_ref, b1_ref, out_ref):
    f = feat_ref[...]
    bsz, m, c = f.shape
    x = f.reshape(bsz * m, c)
    x = jnp.dot(x, wg_ref[...], preferred_element_type=jnp.float32) + bg_ref[...]
    x = jnp.maximum(x, 0.0)
    x = gg_ref[...] * (x / np.sqrt(1.0 + BN_EPS)) + betag_ref[...]
    g = jnp.max(x.reshape(bsz, m, -1), axis=1)
    g = jnp.maximum(g, 0.0)
    g = jnp.maximum(jnp.dot(g, w0_ref[...], preferred_element_type=jnp.float32) + b0_ref[...], 0.0)
    out_ref[...] = jnp.dot(g, w1_ref[...], preferred_element_type=jnp.float32) + b1_ref[...]


def _bn_fold(lyr, pad_to=None):
    inv = 1.0 / np.sqrt(1.0 + BN_EPS)
    s = lyr['gamma'] * inv
    t = lyr['beta']
    if pad_to is not None and s.shape[0] < pad_to:
        p = pad_to - s.shape[0]
        s = jnp.pad(s, (0, p))
        t = jnp.pad(t, (0, p))
    return s, t


def kernel(pos, batch, params):
    del batch
    pos_b = pos.reshape(B, N0, 3)
    px = pos_b[:, :, 0]
    py = pos_b[:, :, 1]
    pz = pos_b[:, :, 2]

    c0x, c0y, c0z = _fps(px, py, pz, M0)
    idx0 = _nbr(px, py, pz, c0x.T, c0y.T, c0z.T, R0, N0, M0)

    l01, l02 = params['mlp0']
    w01 = jnp.pad(l01['W'], ((0, 0), (0, 64)))
    b01 = jnp.pad(l01['b'], (0, 64))
    w02 = jnp.pad(l02['W'], ((0, 64), (0, 0)))
    s01, t01 = _bn_fold(l01, pad_to=128)
    s02, t02 = _bn_fold(l02)

    pxf = px.reshape(B * N0, 1)
    pyf = py.reshape(B * N0, 1)
    pzf = pz.reshape(B * N0, 1)
    p0 = _proj(pxf, pyf, pzf, w01, b01)
    c0xf = c0x.reshape(B * M0, 1)
    c0yf = c0y.reshape(B * M0, 1)
    c0zf = c0z.reshape(B * M0, 1)
    c0p = _proj(c0xf, c0yf, c0zf, w01, jnp.zeros((128,), jnp.float32))

    g0 = _sc_gather(p0, idx0.reshape(-1))
    x1 = _pairmlp(g0.reshape(B * M0, K, 128), c0p.reshape(B * M0, 1, 128),
                  w02, l02['b'], s01, t01, s02, t02, tm=128)

    c1x, c1y, c1z = _fps(c0x, c0y, c0z, M1)
    idx1 = _nbr(c0x, c0y, c0z, c1x.T, c1y.T, c1z.T, R1, M0, M1)

    l11, l12 = params['mlp1']
    wf = l11['W'][:128]
    wp = l11['W'][128:]
    s11, t11 = _bn_fold(l11)
    s12, t12 = _bn_fold(l12)
    q1 = _feat_proj(x1, wf, c0xf, c0yf, c0zf, wp, l11['b'])
    c1xf = c1x.reshape(B * M1, 1)
    c1yf = c1y.reshape(B * M1, 1)
    c1zf = c1z.reshape(B * M1, 1)
    c1p = _proj(c1xf, c1yf, c1zf, wp, jnp.zeros((128,), jnp.float32))

    g1 = _sc_gather(q1, idx1.reshape(-1))
    x2 = _pairmlp(g1.reshape(B * M1, K, 128), c1p.reshape(B * M1, 1, 128),
                  l12['W'], l12['b'], s11, t11, s12, t12, tm=128)

    centers1 = jnp.stack([c1x, c1y, c1z], axis=-1)
    feat = jnp.concatenate([x2.reshape(B, M1, 128), centers1], axis=-1)
    lg = params['mlpg'][0]
    out = pl.pallas_call(
        _head_kernel,
        out_shape=jax.ShapeDtypeStruct((B, 10), jnp.float32),
    )(feat, lg['W'], lg['b'], lg['gamma'], lg['beta'],
      params['lin0']['W'], params['lin0']['b'],
      params['lin1']['W'], params['lin1']['b'])
    return out


# speedup vs baseline: 1.1360x; 1.1360x over previous
import functools
import jax
import jax.numpy as jnp
import numpy as np
from jax.experimental import pallas as pl
from jax.experimental.pallas import tpu as pltpu
from jax.experimental.pallas import tpu_sc as plsc

B = 8
N0 = 2048
M0 = 1024
M1 = 256
K = 64
R0 = 0.2
R1 = 0.4
BN_EPS = 1e-5
INF = jnp.inf


def _fps_body(px_ref, py_ref, pz_ref, cx_ref, cy_ref, cz_ref, *, m):
    px = px_ref[...]
    py = py_ref[...]
    pz = pz_ref[...]
    n = px.shape[1]
    lane_n = jax.lax.broadcasted_iota(jnp.int32, (B, n), 1)
    lane_m = jax.lax.broadcasted_iota(jnp.int32, (B, m), 1)

    def body(i, carry):
        dists, cur, ax, ay, az = carry
        mask = lane_n == cur
        cx = jnp.max(jnp.where(mask, px, -INF), axis=1, keepdims=True)
        cy = jnp.max(jnp.where(mask, py, -INF), axis=1, keepdims=True)
        cz = jnp.max(jnp.where(mask, pz, -INF), axis=1, keepdims=True)
        smask = lane_m == i
        ax = jnp.where(smask, cx, ax)
        ay = jnp.where(smask, cy, ay)
        az = jnp.where(smask, cz, az)
        d = (px - cx) ** 2 + (py - cy) ** 2 + (pz - cz) ** 2
        dists = jnp.minimum(dists, d)
        mx = jnp.max(dists, axis=1, keepdims=True)
        cur = jnp.min(jnp.where(dists == mx, lane_n, n), axis=1, keepdims=True)
        return dists, cur, ax, ay, az

    init = (
        jnp.full((B, n), INF, jnp.float32),
        jnp.zeros((B, 1), jnp.int32),
        jnp.zeros((B, m), jnp.float32),
        jnp.zeros((B, m), jnp.float32),
        jnp.zeros((B, m), jnp.float32),
    )
    _, _, ax, ay, az = jax.lax.fori_loop(0, m, body, init)
    cx_ref[...] = ax
    cy_ref[...] = ay
    cz_ref[...] = az


def _fps(px, py, pz, m):
    out = jax.ShapeDtypeStruct((B, m), jnp.float32)
    return pl.pallas_call(
        functools.partial(_fps_body, m=m),
        out_shape=(out, out, out),
    )(px, py, pz)


def _nbr_body(px_ref, py_ref, pz_ref, cx_ref, cy_ref, cz_ref, pack_ref,
              idx_ref, *, r2, n, m):
    b = pl.program_id(0)
    px = px_ref[pl.ds(b, 1), :]
    py = py_ref[pl.ds(b, 1), :]
    pz = pz_ref[pl.ds(b, 1), :]
    lane_b = jax.lax.broadcasted_iota(jnp.int32, (m, B), 1)
    colmask = lane_b == b
    cx = jnp.sum(jnp.where(colmask, cx_ref[...], 0.0), axis=1, keepdims=True)
    cy = jnp.sum(jnp.where(colmask, cy_ref[...], 0.0), axis=1, keepdims=True)
    cz = jnp.sum(jnp.where(colmask, cz_ref[...], 0.0), axis=1, keepdims=True)
    d2 = (cx - px) ** 2 + (cy - py) ** 2 + (cz - pz) ** 2
    d2 = jnp.where(d2 <= r2, d2, INF)

    def rbody(i, t):
        bb = 29 - i
        u = t | (jax.lax.shift_left(jnp.int32(1), bb) - 1)
        uf = jax.lax.bitcast_convert_type(u, jnp.float32)
        cnt = jnp.sum(jnp.where(d2 <= uf, 1.0, 0.0), axis=1, keepdims=True)
        return jnp.where(cnt >= float(K), t,
                         t | jax.lax.shift_left(jnp.int32(1), bb))

    t = jax.lax.fori_loop(0, 30, rbody, jnp.zeros((m, 1), jnp.int32))
    tf = jax.lax.bitcast_convert_type(t, jnp.float32)
    selm = jnp.where(d2 <= tf, 1.0, 0.0)
    words = jnp.dot(selm, pack_ref[...],
                    preferred_element_type=jnp.float32).astype(jnp.int32)
    nw = n // 16
    lane_w = jax.lax.broadcasted_iota(jnp.int32, (m, nw), 1)
    lane_k = jax.lax.broadcasted_iota(jnp.int32, (m, K), 1)

    def ebody(s, carry):
        words, first, acc = carry
        nz = words != 0
        fl = jnp.min(jnp.where(nz, lane_w, nw), axis=1, keepdims=True)
        has = fl < nw
        onfl = lane_w == fl
        w = jnp.max(jnp.where(onfl, words, 0), axis=1, keepdims=True)
        low = jnp.bitwise_and(w, -w)
        bidx = jax.lax.shift_right_logical(
            jax.lax.bitcast_convert_type(low.astype(jnp.float32), jnp.int32),
            23) - 127
        idxv = fl * 16 + bidx
        first = jnp.where(s == 0, idxv, first)
        selv = jnp.where(has, idxv, first)
        acc = jnp.where(lane_k == s, selv, acc)
        words = jnp.where(onfl, jnp.bitwise_and(words, words - 1), words)
        return words, first, acc

    _, _, acc = jax.lax.fori_loop(
        0, K, ebody,
        (words, jnp.zeros((m, 1), jnp.int32), jnp.zeros((m, K), jnp.int32)))
    idx_ref[0] = acc + b * n


def _nbr(px, py, pz, cxt, cyt, czt, r, n, m):
    nw = n // 16
    packw = np.zeros((n, nw), np.float32)
    packw[np.arange(n), np.arange(n) // 16] = 2.0 ** (np.arange(n) % 16)
    return pl.pallas_call(
        functools.partial(_nbr_body, r2=np.float32(r * r), n=n, m=m),
        grid=(B,),
        in_specs=[
            pl.BlockSpec((B, n), lambda b: (0, 0)),
            pl.BlockSpec((B, n), lambda b: (0, 0)),
            pl.BlockSpec((B, n), lambda b: (0, 0)),
            pl.BlockSpec((m, B), lambda b: (0, 0)),
            pl.BlockSpec((m, B), lambda b: (0, 0)),
            pl.BlockSpec((m, B), lambda b: (0, 0)),
            pl.BlockSpec((n, nw), lambda b: (0, 0)),
        ],
        out_specs=pl.BlockSpec((1, m, K), lambda b: (b, 0, 0)),
        out_shape=jax.ShapeDtypeStruct((B, m, K), jnp.int32),
    )(px, py, pz, cxt, cyt, czt, jnp.asarray(packw))


def _sc_gather(table, indices):
    num = indices.shape[0]
    cdim = table.shape[1]
    inds = indices.reshape(1, num)
    mesh = plsc.VectorSubcoreMesh(core_axis_name="c", subcore_axis_name="s")
    win = 256

    @pl.kernel(out_type=jax.ShapeDtypeStruct((num, cdim), table.dtype),
               mesh=mesh)
    def kern(x_hbm, i_hbm, o_hbm):
        def body(i_vmem, o_vmem):
            pltpu.sync_copy(x_hbm.at[i_vmem.at[0]], o_vmem)

        pltpu.emit_pipeline(
            body,
            grid=(num // win,),
            in_specs=[pl.BlockSpec((1, win), index_map=lambda i: (0, i))],
            out_specs=[pl.BlockSpec((win, cdim), index_map=lambda i: (i, 0))],
            core_axis_name="s",
            dimension_semantics=(pltpu.PARALLEL,),
        )(i_hbm, o_hbm)

    return kern(table, inds)


def _proj_body(x_ref, y_ref, z_ref, w_ref, b_ref, out_ref):
    w = w_ref[...]
    acc = (x_ref[...] * w[0:1, :] + y_ref[...] * w[1:2, :]
           + z_ref[...] * w[2:3, :] + b_ref[...])
    out_ref[...] = acc


def _proj(xc, yc, zc, w, b):
    n = xc.shape[0]
    c = w.shape[1]
    return pl.pallas_call(
        _proj_body,
        out_shape=jax.ShapeDtypeStruct((n, c), jnp.float32),
    )(xc, yc, zc, w, b.reshape(1, c))


def _feat_proj_body(x_ref, w_ref, px_ref, py_ref, pz_ref, wp_ref, b_ref,
                    out_ref):
    wp = wp_ref[...]
    acc = jnp.dot(x_ref[...], w_ref[...], preferred_element_type=jnp.float32)
    acc += (px_ref[...] * wp[0:1, :] + py_ref[...] * wp[1:2, :]
            + pz_ref[...] * wp[2:3, :] + b_ref[...])
    out_ref[...] = acc


def _feat_proj(x, w, px, py, pz, wp, b):
    n, c = x.shape[0], w.shape[1]
    return pl.pallas_call(
        _feat_proj_body,
        out_shape=jax.ShapeDtypeStruct((n, c), jnp.float32),
    )(x, w, px, py, pz, wp, b.reshape(1, c))


def _pairmlp(g3, c3, w2, b2, s1, t1, s2, t2, tm):
    m, k, ch = g3.shape
    c2 = w2.shape[1]

    def body(g_ref, c_ref, w2_ref, b2_ref, s1_ref, t1_ref, s2_ref, t2_ref,
             out_ref):
        g = g_ref[...]
        c = c_ref[...]
        h = jnp.maximum(g - c, 0.0) * s1_ref[...] + t1_ref[...]
        h2 = jnp.dot(h.reshape(tm * k, ch), w2_ref[...],
                     preferred_element_type=jnp.float32) + b2_ref[...]
        h2 = jnp.maximum(h2, 0.0) * s2_ref[...] + t2_ref[...]
        out_ref[...] = jnp.max(h2.reshape(tm, k, c2), axis=1)

    return pl.pallas_call(
        body,
        grid=(m // tm,),
        in_specs=[
            pl.BlockSpec((tm, k, ch), lambda i: (i, 0, 0)),
            pl.BlockSpec((tm, 1, ch), lambda i: (i, 0, 0)),
            pl.BlockSpec((ch, c2), lambda i: (0, 0)),
            pl.BlockSpec((1, c2), lambda i: (0, 0)),
            pl.BlockSpec((1, 1, ch), lambda i: (0, 0, 0)),
            pl.BlockSpec((1, 1, ch), lambda i: (0, 0, 0)),
            pl.BlockSpec((1, c2), lambda i: (0, 0)),
            pl.BlockSpec((1, c2), lambda i: (0, 0)),
        ],
        out_specs=pl.BlockSpec((tm, c2), lambda i: (i, 0)),
        out_shape=jax.ShapeDtypeStruct((m, c2), jnp.float32),
    )(g3, c3, w2, b2.reshape(1, c2), s1.reshape(1, 1, ch),
      t1.reshape(1, 1, ch), s2.reshape(1, c2), t2.reshape(1, c2))


def _head_kernel(feat_ref, wg_ref, bg_ref, gg_ref, betag_ref,
                 w0_ref, b0_ref, w1_ref, b1_ref, out_ref):
    f = feat_ref[...]
    bsz, m, c = f.shape
    x = f.reshape(bsz * m, c)
    x = jnp.dot(x, wg_ref[...], preferred_element_type=jnp.float32) + bg_ref[...]
    x = jnp.maximum(x, 0.0)
    x = gg_ref[...] * (x / np.sqrt(1.0 + BN_EPS)) + betag_ref[...]
    g = jnp.max(x.reshape(bsz, m, -1), axis=1)
    g = jnp.maximum(g, 0.0)
    g = jnp.maximum(jnp.dot(g, w0_ref[...], preferred_element_type=jnp.float32) + b0_ref[...], 0.0)
    out_ref[...] = jnp.dot(g, w1_ref[...], preferred_element_type=jnp.float32) + b1_ref[...]


def _bn_fold(lyr, pad_to=None):
    inv = 1.0 / np.sqrt(1.0 + BN_EPS)
    s = lyr['gamma'] * inv
    t = lyr['beta']
    if pad_to is not None and s.shape[0] < pad_to:
        p = pad_to - s.shape[0]
        s = jnp.pad(s, (0, p))
        t = jnp.pad(t, (0, p))
    return s, t


def kernel(pos, batch, params):
    del batch
    pos_b = pos.reshape(B, N0, 3)
    px = pos_b[:, :, 0]
    py = pos_b[:, :, 1]
    pz = pos_b[:, :, 2]

    c0x, c0y, c0z = _fps(px, py, pz, M0)
    idx0 = _nbr(px, py, pz, c0x.T, c0y.T, c0z.T, R0, N0, M0)

    l01, l02 = params['mlp0']
    w01 = jnp.pad(l01['W'], ((0, 0), (0, 64)))
    b01 = jnp.pad(l01['b'], (0, 64))
    w02 = jnp.pad(l02['W'], ((0, 64), (0, 0)))
    s01, t01 = _bn_fold(l01, pad_to=128)
    s02, t02 = _bn_fold(l02)

    pxf = px.reshape(B * N0, 1)
    pyf = py.reshape(B * N0, 1)
    pzf = pz.reshape(B * N0, 1)
    p0 = _proj(pxf, pyf, pzf, w01, b01)
    c0xf = c0x.reshape(B * M0, 1)
    c0yf = c0y.reshape(B * M0, 1)
    c0zf = c0z.reshape(B * M0, 1)
    c0p = _proj(c0xf, c0yf, c0zf, w01, jnp.zeros((128,), jnp.float32))

    g0 = _sc_gather(p0, idx0.reshape(-1))
    x1 = _pairmlp(g0.reshape(B * M0, K, 128), c0p.reshape(B * M0, 1, 128),
                  w02, l02['b'], s01, t01, s02, t02, tm=128)

    c1x, c1y, c1z = _fps(c0x, c0y, c0z, M1)
    idx1 = _nbr(c0x, c0y, c0z, c1x.T, c1y.T, c1z.T, R1, M0, M1)

    l11, l12 = params['mlp1']
    wf = l11['W'][:128]
    wp = l11['W'][128:]
    s11, t11 = _bn_fold(l11)
    s12, t12 = _bn_fold(l12)
    q1 = _feat_proj(x1, wf, c0xf, c0yf, c0zf, wp, l11['b'])
    c1xf = c1x.reshape(B * M1, 1)
    c1yf = c1y.reshape(B * M1, 1)
    c1zf = c1z.reshape(B * M1, 1)
    c1p = _proj(c1xf, c1yf, c1zf, wp, jnp.zeros((128,), jnp.float32))

    g1 = _sc_gather(q1, idx1.reshape(-1))
    x2 = _pairmlp(g1.reshape(B * M1, K, 128), c1p.reshape(B * M1, 1, 128),
                  l12['W'], l12['b'], s11, t11, s12, t12, tm=128)

    centers1 = jnp.stack([c1x, c1y, c1z], axis=-1)
    feat = jnp.concatenate([x2.reshape(B, M1, 128), centers1], axis=-1)
    lg = params['mlpg'][0]
    out = pl.pallas_call(
        _head_kernel,
        out_shape=jax.ShapeDtypeStruct((B, 10), jnp.float32),
    )(feat, lg['W'], lg['b'], lg['gamma'], lg['beta'],
      params['lin0']['W'], params['lin0']['b'],
      params['lin1']['W'], params['lin1']['b'])
    return out


# speedup vs baseline: 1.4225x; 1.2522x over previous
import functools
import jax
import jax.numpy as jnp
import numpy as np
from jax.experimental import pallas as pl
from jax.experimental.pallas import tpu as pltpu
from jax.experimental.pallas import tpu_sc as plsc

B = 8
N0 = 2048
M0 = 1024
M1 = 256
K = 64
R0 = 0.2
R1 = 0.4
BN_EPS = 1e-5
INF = jnp.inf


def _fps_body(px_ref, py_ref, pz_ref, cx_ref, cy_ref, cz_ref, *, m):
    px = px_ref[...]
    py = py_ref[...]
    pz = pz_ref[...]
    n = px.shape[1]
    lane_n = jax.lax.broadcasted_iota(jnp.int32, (B, n), 1)
    lane_m = jax.lax.broadcasted_iota(jnp.int32, (B, m), 1)

    def body(i, carry):
        dists, cur, ax, ay, az = carry
        mask = lane_n == cur
        cx = jnp.max(jnp.where(mask, px, -INF), axis=1, keepdims=True)
        cy = jnp.max(jnp.where(mask, py, -INF), axis=1, keepdims=True)
        cz = jnp.max(jnp.where(mask, pz, -INF), axis=1, keepdims=True)
        smask = lane_m == i
        ax = jnp.where(smask, cx, ax)
        ay = jnp.where(smask, cy, ay)
        az = jnp.where(smask, cz, az)
        d = (px - cx) ** 2 + (py - cy) ** 2 + (pz - cz) ** 2
        dists = jnp.minimum(dists, d)
        mx = jnp.max(dists, axis=1, keepdims=True)
        cur = jnp.min(jnp.where(dists == mx, lane_n, n), axis=1, keepdims=True)
        return dists, cur, ax, ay, az

    init = (
        jnp.full((B, n), INF, jnp.float32),
        jnp.zeros((B, 1), jnp.int32),
        jnp.zeros((B, m), jnp.float32),
        jnp.zeros((B, m), jnp.float32),
        jnp.zeros((B, m), jnp.float32),
    )
    _, _, ax, ay, az = jax.lax.fori_loop(0, m, body, init)
    cx_ref[...] = ax
    cy_ref[...] = ay
    cz_ref[...] = az


def _fps(px, py, pz, m):
    out = jax.ShapeDtypeStruct((B, m), jnp.float32)
    return pl.pallas_call(
        functools.partial(_fps_body, m=m),
        out_shape=(out, out, out),
    )(px, py, pz)


def _nbr_body(px_ref, py_ref, pz_ref, cx_ref, cy_ref, cz_ref, pack_ref,
              idx_ref, *, r2, n, m):
    b = pl.program_id(0)
    px = px_ref[pl.ds(b, 1), :]
    py = py_ref[pl.ds(b, 1), :]
    pz = pz_ref[pl.ds(b, 1), :]
    lane_b = jax.lax.broadcasted_iota(jnp.int32, (m, B), 1)
    colmask = lane_b == b
    cx = jnp.sum(jnp.where(colmask, cx_ref[...], 0.0), axis=1, keepdims=True)
    cy = jnp.sum(jnp.where(colmask, cy_ref[...], 0.0), axis=1, keepdims=True)
    cz = jnp.sum(jnp.where(colmask, cz_ref[...], 0.0), axis=1, keepdims=True)
    d2 = (cx - px) ** 2 + (cy - py) ** 2 + (cz - pz) ** 2
    d2 = jnp.where(d2 <= r2, d2, INF)

    def rbody(i, t):
        bb = 29 - i
        u = t | (jax.lax.shift_left(jnp.int32(1), bb) - 1)
        uf = jax.lax.bitcast_convert_type(u, jnp.float32)
        cnt = jnp.sum(jnp.where(d2 <= uf, 1.0, 0.0), axis=1, keepdims=True)
        return jnp.where(cnt >= float(K), t,
                         t | jax.lax.shift_left(jnp.int32(1), bb))

    t = jax.lax.fori_loop(0, 30, rbody, jnp.zeros((m, 1), jnp.int32))
    tf = jax.lax.bitcast_convert_type(t, jnp.float32)
    selm = jnp.where(d2 <= tf, 1.0, 0.0)
    words = jnp.dot(selm, pack_ref[...],
                    preferred_element_type=jnp.float32).astype(jnp.int32)
    nw = n // 16
    lane_w = jax.lax.broadcasted_iota(jnp.int32, (m, nw), 1)
    lane_k = jax.lax.broadcasted_iota(jnp.int32, (m, K), 1)

    def ebody(s, carry):
        words, first, acc = carry
        nz = words != 0
        fl = jnp.min(jnp.where(nz, lane_w, nw), axis=1, keepdims=True)
        has = fl < nw
        onfl = lane_w == fl
        w = jnp.max(jnp.where(onfl, words, 0), axis=1, keepdims=True)
        low = jnp.bitwise_and(w, -w)
        bidx = jax.lax.shift_right_logical(
            jax.lax.bitcast_convert_type(low.astype(jnp.float32), jnp.int32),
            23) - 127
        idxv = fl * 16 + bidx
        first = jnp.where(s == 0, idxv, first)
        selv = jnp.where(has, idxv, first)
        acc = jnp.where(lane_k == s, selv, acc)
        words = jnp.where(onfl, jnp.bitwise_and(words, words - 1), words)
        return words, first, acc

    _, _, acc = jax.lax.fori_loop(
        0, K, ebody,
        (words, jnp.zeros((m, 1), jnp.int32), jnp.zeros((m, K), jnp.int32)))
    idx_ref[0] = acc + b * n


def _nbr(px, py, pz, cxt, cyt, czt, r, n, m):
    nw = n // 16
    packw = np.zeros((n, nw), np.float32)
    packw[np.arange(n), np.arange(n) // 16] = 2.0 ** (np.arange(n) % 16)
    return pl.pallas_call(
        functools.partial(_nbr_body, r2=np.float32(r * r), n=n, m=m),
        grid=(B,),
        in_specs=[
            pl.BlockSpec((B, n), lambda b: (0, 0)),
            pl.BlockSpec((B, n), lambda b: (0, 0)),
            pl.BlockSpec((B, n), lambda b: (0, 0)),
            pl.BlockSpec((m, B), lambda b: (0, 0)),
            pl.BlockSpec((m, B), lambda b: (0, 0)),
            pl.BlockSpec((m, B), lambda b: (0, 0)),
            pl.BlockSpec((n, nw), lambda b: (0, 0)),
        ],
        out_specs=pl.BlockSpec((1, m, K), lambda b: (b, 0, 0)),
        out_shape=jax.ShapeDtypeStruct((B, m, K), jnp.int32),
    )(px, py, pz, cxt, cyt, czt, jnp.asarray(packw))


def _sc_gather(table, indices):
    num = indices.shape[0]
    cdim = table.shape[1]
    inds = indices.reshape(1, num)
    mesh = plsc.VectorSubcoreMesh(core_axis_name="c", subcore_axis_name="s")
    win = 256

    @pl.kernel(out_type=jax.ShapeDtypeStruct((num, cdim), table.dtype),
               mesh=mesh)
    def kern(x_hbm, i_hbm, o_hbm):
        def body(i_vmem, o_vmem):
            pltpu.sync_copy(x_hbm.at[i_vmem.at[0]], o_vmem)

        pltpu.emit_pipeline(
            body,
            grid=(num // win,),
            in_specs=[pl.BlockSpec((1, win), index_map=lambda i: (0, i))],
            out_specs=[pl.BlockSpec((win, cdim), index_map=lambda i: (i, 0))],
            core_axis_name=("c", "s"),
            dimension_semantics=(pltpu.PARALLEL,),
        )(i_hbm, o_hbm)

    return kern(table, inds)


def _proj_body(x_ref, y_ref, z_ref, w_ref, b_ref, out_ref):
    w = w_ref[...]
    acc = (x_ref[...] * w[0:1, :] + y_ref[...] * w[1:2, :]
           + z_ref[...] * w[2:3, :] + b_ref[...])
    out_ref[...] = acc


def _proj(xc, yc, zc, w, b):
    n = xc.shape[0]
    c = w.shape[1]
    return pl.pallas_call(
        _proj_body,
        out_shape=jax.ShapeDtypeStruct((n, c), jnp.float32),
    )(xc, yc, zc, w, b.reshape(1, c))


def _feat_proj_body(x_ref, w_ref, px_ref, py_ref, pz_ref, wp_ref, b_ref,
                    out_ref):
    wp = wp_ref[...]
    acc = jnp.dot(x_ref[...], w_ref[...], preferred_element_type=jnp.float32)
    acc += (px_ref[...] * wp[0:1, :] + py_ref[...] * wp[1:2, :]
            + pz_ref[...] * wp[2:3, :] + b_ref[...])
    out_ref[...] = acc


def _feat_proj(x, w, px, py, pz, wp, b):
    n, c = x.shape[0], w.shape[1]
    return pl.pallas_call(
        _feat_proj_body,
        out_shape=jax.ShapeDtypeStruct((n, c), jnp.float32),
    )(x, w, px, py, pz, wp, b.reshape(1, c))


def _pairmlp(g3, c3, w2, b2, s1, t1, s2, t2, tm):
    m, k, ch = g3.shape
    c2 = w2.shape[1]

    def body(g_ref, c_ref, w2_ref, b2_ref, s1_ref, t1_ref, s2_ref, t2_ref,
             out_ref):
        g = g_ref[...]
        c = c_ref[...]
        h = jnp.maximum(g - c, 0.0) * s1_ref[...] + t1_ref[...]
        h2 = jnp.dot(h.reshape(tm * k, ch), w2_ref[...],
                     preferred_element_type=jnp.float32) + b2_ref[...]
        h2 = jnp.maximum(h2, 0.0) * s2_ref[...] + t2_ref[...]
        out_ref[...] = jnp.max(h2.reshape(tm, k, c2), axis=1)

    return pl.pallas_call(
        body,
        grid=(m // tm,),
        in_specs=[
            pl.BlockSpec((tm, k, ch), lambda i: (i, 0, 0)),
            pl.BlockSpec((tm, 1, ch), lambda i: (i, 0, 0)),
            pl.BlockSpec((ch, c2), lambda i: (0, 0)),
            pl.BlockSpec((1, c2), lambda i: (0, 0)),
            pl.BlockSpec((1, 1, ch), lambda i: (0, 0, 0)),
            pl.BlockSpec((1, 1, ch), lambda i: (0, 0, 0)),
            pl.BlockSpec((1, c2), lambda i: (0, 0)),
            pl.BlockSpec((1, c2), lambda i: (0, 0)),
        ],
        out_specs=pl.BlockSpec((tm, c2), lambda i: (i, 0)),
        out_shape=jax.ShapeDtypeStruct((m, c2), jnp.float32),
    )(g3, c3, w2, b2.reshape(1, c2), s1.reshape(1, 1, ch),
      t1.reshape(1, 1, ch), s2.reshape(1, c2), t2.reshape(1, c2))


def _head_kernel(feat_ref, wg_ref, bg_ref, gg_ref, betag_ref,
                 w0_ref, b0_ref, w1_ref, b1_ref, out_ref):
    f = feat_ref[...]
    bsz, m, c = f.shape
    x = f.reshape(bsz * m, c)
    x = jnp.dot(x, wg_ref[...], preferred_element_type=jnp.float32) + bg_ref[...]
    x = jnp.maximum(x, 0.0)
    x = gg_ref[...] * (x / np.sqrt(1.0 + BN_EPS)) + betag_ref[...]
    g = jnp.max(x.reshape(bsz, m, -1), axis=1)
    g = jnp.maximum(g, 0.0)
    g = jnp.maximum(jnp.dot(g, w0_ref[...], preferred_element_type=jnp.float32) + b0_ref[...], 0.0)
    out_ref[...] = jnp.dot(g, w1_ref[...], preferred_element_type=jnp.float32) + b1_ref[...]


def _bn_fold(lyr, pad_to=None):
    inv = 1.0 / np.sqrt(1.0 + BN_EPS)
    s = lyr['gamma'] * inv
    t = lyr['beta']
    if pad_to is not None and s.shape[0] < pad_to:
        p = pad_to - s.shape[0]
        s = jnp.pad(s, (0, p))
        t = jnp.pad(t, (0, p))
    return s, t


def kernel(pos, batch, params):
    del batch
    pos_b = pos.reshape(B, N0, 3)
    px = pos_b[:, :, 0]
    py = pos_b[:, :, 1]
    pz = pos_b[:, :, 2]

    c0x, c0y, c0z = _fps(px, py, pz, M0)
    idx0 = _nbr(px, py, pz, c0x.T, c0y.T, c0z.T, R0, N0, M0)

    l01, l02 = params['mlp0']
    w01 = jnp.pad(l01['W'], ((0, 0), (0, 64)))
    b01 = jnp.pad(l01['b'], (0, 64))
    w02 = jnp.pad(l02['W'], ((0, 64), (0, 0)))
    s01, t01 = _bn_fold(l01, pad_to=128)
    s02, t02 = _bn_fold(l02)

    pxf = px.reshape(B * N0, 1)
    pyf = py.reshape(B * N0, 1)
    pzf = pz.reshape(B * N0, 1)
    p0 = _proj(pxf, pyf, pzf, w01, b01)
    c0xf = c0x.reshape(B * M0, 1)
    c0yf = c0y.reshape(B * M0, 1)
    c0zf = c0z.reshape(B * M0, 1)
    c0p = _proj(c0xf, c0yf, c0zf, w01, jnp.zeros((128,), jnp.float32))

    g0 = _sc_gather(p0, idx0.reshape(-1))
    x1 = _pairmlp(g0.reshape(B * M0, K, 128), c0p.reshape(B * M0, 1, 128),
                  w02, l02['b'], s01, t01, s02, t02, tm=128)

    c1x, c1y, c1z = _fps(c0x, c0y, c0z, M1)
    idx1 = _nbr(c0x, c0y, c0z, c1x.T, c1y.T, c1z.T, R1, M0, M1)

    l11, l12 = params['mlp1']
    wf = l11['W'][:128]
    wp = l11['W'][128:]
    s11, t11 = _bn_fold(l11)
    s12, t12 = _bn_fold(l12)
    q1 = _feat_proj(x1, wf, c0xf, c0yf, c0zf, wp, l11['b'])
    c1xf = c1x.reshape(B * M1, 1)
    c1yf = c1y.reshape(B * M1, 1)
    c1zf = c1z.reshape(B * M1, 1)
    c1p = _proj(c1xf, c1yf, c1zf, wp, jnp.zeros((128,), jnp.float32))

    g1 = _sc_gather(q1, idx1.reshape(-1))
    x2 = _pairmlp(g1.reshape(B * M1, K, 128), c1p.reshape(B * M1, 1, 128),
                  l12['W'], l12['b'], s11, t11, s12, t12, tm=128)

    centers1 = jnp.stack([c1x, c1y, c1z], axis=-1)
    feat = jnp.concatenate([x2.reshape(B, M1, 128), centers1], axis=-1)
    lg = params['mlpg'][0]
    out = pl.pallas_call(
        _head_kernel,
        out_shape=jax.ShapeDtypeStruct((B, 10), jnp.float32),
    )(feat, lg['W'], lg['b'], lg['gamma'], lg['beta'],
      params['lin0']['W'], params['lin0']['b'],
      params['lin1']['W'], params['lin1']['b'])
    return out
